# baseline (speedup 1.0000x reference)
import functools

import jax
import jax.numpy as jnp
from jax import lax
from jax.experimental import pallas as pl
from jax.experimental.pallas import tpu as pltpu
from jax.experimental.pallas import tpu_sc as plsc

_NC = 2
_NS = 16
_NW = _NC * _NS

_NB = 4096
_T = 200
_D = 64
_DP = 128
_V = 8192
_B = _NB * _T
_BPW = _B // _NW
_CH = 80
_NCHUNK = _BPW // _CH

_mesh = plsc.VectorSubcoreMesh(core_axis_name="c", subcore_axis_name="s")


@functools.partial(
    pl.kernel,
    mesh=_mesh,
    out_type=jax.ShapeDtypeStruct((_B, _DP), jnp.float32),
    scratch_types=[
        [pltpu.VMEM((_CH,), jnp.int32) for _ in range(4)],
        [pltpu.VMEM((_CH, _DP), jnp.float32) for _ in range(4)],
        pltpu.VMEM_SHARED((_V, _DP), jnp.float32),
        [pltpu.SemaphoreType.DMA for _ in range(4)],
        [pltpu.SemaphoreType.DMA for _ in range(4)],
        [pltpu.SemaphoreType.DMA for _ in range(4)],
    ],
)
def _gather_kernel(idx_hbm, table_hbm, out_hbm, idx_v, rows, table_s,
                   isem, gsem, osem):
    wid = lax.axis_index("s") * _NC + lax.axis_index("c")
    base = wid * _BPW

    sid = lax.axis_index("s")
    stripe = _V // _NS
    pltpu.sync_copy(table_hbm.at[pl.ds(sid * stripe, stripe)],
                    table_s.at[pl.ds(sid * stripe, stripe)])
    plsc.subcore_barrier()

    def _idx(i, q):
        pltpu.async_copy(idx_hbm.at[pl.ds(base + i * _CH, _CH)], idx_v[q],
                         isem[q])

    def _idx_wait(i, q):
        pltpu.make_async_copy(idx_hbm.at[pl.ds(base + i * _CH, _CH)],
                              idx_v[q], isem[q]).wait()

    def _gather(q):
        pltpu.async_copy(table_s.at[idx_v[q]], rows[q], gsem[q])

    def _gather_wait(q):
        pltpu.make_async_copy(table_s.at[idx_v[q]], rows[q], gsem[q]).wait()

    def _store(i, q):
        pltpu.async_copy(rows[q], out_hbm.at[pl.ds(base + i * _CH, _CH)],
                         osem[q])

    def _store_wait(i, q):
        pltpu.make_async_copy(
            rows[q], out_hbm.at[pl.ds(base + i * _CH, _CH)], osem[q]).wait()

    for q in range(4):
        _idx(q, q)
    for q in range(4):
        _idx_wait(q, q)
        _gather(q)

    def outer(k, carry):
        i0 = k * 4
        for q in range(4):
            i = i0 + q
            _gather_wait(q)
            _store(i, q)
            j = i + 4

            @pl.when(j < _NCHUNK)
            def _():
                _idx(j, q)
                _store_wait(i, q)
                _idx_wait(j, q)
                _gather(q)
        return carry

    lax.fori_loop(0, _NCHUNK // 4, outer, 0)

    for q in range(4):
        _store_wait(_NCHUNK - 4 + q, q)


def kernel(tokens, embedding_table):
    idx = tokens.reshape(-1).astype(jnp.int32)
    table_p = jnp.pad(embedding_table, ((0, 0), (0, _DP - _D)))
    out = _gather_kernel(idx, table_p)
    return out[:, :_D].reshape(tokens.shape + (_D,))


# speedup vs baseline: 1.0634x; 1.0634x over previous
import functools

import jax
import jax.numpy as jnp
from jax import lax
from jax.experimental import pallas as pl
from jax.experimental.pallas import tpu as pltpu
from jax.experimental.pallas import tpu_sc as plsc

_NC = 2
_NS = 16
_NW = _NC * _NS

_NB = 4096
_T = 200
_D = 64
_DP = 128
_V = 8192
_B = _NB * _T
_BPW = _B // _NW
_CH = 200
_NCHUNK = _BPW // _CH

_mesh = plsc.VectorSubcoreMesh(core_axis_name="c", subcore_axis_name="s")


@functools.partial(
    pl.kernel,
    mesh=_mesh,
    out_type=jax.ShapeDtypeStruct((_B, _DP), jnp.float32),
    scratch_types=[
        [pltpu.VMEM((_CH,), jnp.int32) for _ in range(2)],
        [pltpu.VMEM((_CH, _DP), jnp.float32) for _ in range(2)],
        pltpu.VMEM_SHARED((_V, _DP), jnp.float32),
        [pltpu.SemaphoreType.DMA for _ in range(2)],
        [pltpu.SemaphoreType.DMA for _ in range(2)],
        [pltpu.SemaphoreType.DMA for _ in range(2)],
    ],
)
def _gather_kernel(idx_hbm, table_hbm, out_hbm, idx_v, rows, table_s,
                   isem, gsem, osem):
    wid = lax.axis_index("s") * _NC + lax.axis_index("c")
    base = wid * _BPW

    sid = lax.axis_index("s")
    stripe = _V // _NS
    pltpu.sync_copy(table_hbm.at[pl.ds(sid * stripe, stripe)],
                    table_s.at[pl.ds(sid * stripe, stripe)])
    plsc.subcore_barrier()

    def _idx(i, q):
        pltpu.async_copy(idx_hbm.at[pl.ds(base + i * _CH, _CH)], idx_v[q],
                         isem[q])

    def _idx_wait(i, q):
        pltpu.make_async_copy(idx_hbm.at[pl.ds(base + i * _CH, _CH)],
                              idx_v[q], isem[q]).wait()

    def _gather(q):
        pltpu.async_copy(table_s.at[idx_v[q]], rows[q], gsem[q])

    def _gather_wait(q):
        pltpu.make_async_copy(table_s.at[idx_v[q]], rows[q], gsem[q]).wait()

    def _store(i, q):
        pltpu.async_copy(rows[q], out_hbm.at[pl.ds(base + i * _CH, _CH)],
                         osem[q])

    def _store_wait(i, q):
        pltpu.make_async_copy(
            rows[q], out_hbm.at[pl.ds(base + i * _CH, _CH)], osem[q]).wait()

    for q in range(2):
        _idx(q, q)
    for q in range(2):
        _idx_wait(q, q)
        _gather(q)

    def outer(k, carry):
        i0 = k * 2
        for q in range(2):
            i = i0 + q
            _gather_wait(q)
            _store(i, q)
            j = i + 2

            @pl.when(j < _NCHUNK)
            def _():
                _idx(j, q)
                _store_wait(i, q)
                _idx_wait(j, q)
                _gather(q)
        return carry

    lax.fori_loop(0, _NCHUNK // 2, outer, 0)

    for q in range(2):
        _store_wait(_NCHUNK - 2 + q, q)


def kernel(tokens, embedding_table):
    idx = tokens.reshape(-1).astype(jnp.int32)
    table_p = jnp.pad(embedding_table, ((0, 0), (0, _DP - _D)))
    out = _gather_kernel(idx, table_p)
    return out[:, :_D].reshape(tokens.shape + (_D,))
